# baseline (speedup 1.0000x reference)
import jax
import jax.numpy as jnp
from jax import lax
from jax.experimental import pallas as pl
from jax.experimental.pallas import tpu as pltpu
from jax.experimental.pallas import tpu_sc as plsc

NC = 2
NS = 16
L = 16
NW = NC * NS

B = 16384
D = 256
W2 = D // 2
NA = 32
E = 128
BPW = B // NW
NCHUNK = BPW // E
NBW = W2 // L

def _body(stm, nstm, fstm, fnstm, ft, fft, bias, w, outb, out,
          i0, i1, i2, i3, rows, pbuf, bias_v, w_v, outb_v, outc,
          semA, semB):
    wid = lax.axis_index("s") * NC + lax.axis_index("c")

    pltpu.sync_copy(bias, bias_v)
    pltpu.sync_copy(w, w_v)
    pltpu.sync_copy(outb, outb_v)

    lane = lax.iota(jnp.int32, L)

    def fire2(e, par, sem, big, small):
        pltpu.async_copy(ft.at[big.at[e]], rows.at[par, pl.ds(0, NA)], sem)
        pltpu.async_copy(fft.at[small.at[e]], rows.at[par, pl.ds(NA, NA)], sem)

    def drain2(par, sem):
        for _ in range(2):
            pltpu.make_async_copy(
                ft.at[i0.at[0]], rows.at[par, pl.ds(0, NA)], sem).wait()

    def unpack(wv):
        lo = plsc.bitcast(lax.shift_left(wv, 16), jnp.float32)
        hi = plsc.bitcast(jnp.bitwise_and(wv, -65536), jnp.float32)
        return lo, hi

    def reduce_half(par, half, p):
        def rbody(j, ss):
            r0 = 4 * j
            out_ss = []
            for m in range(NBW):
                lo = []
                hi = []
                for dr in range(4):
                    a, b = unpack(rows[par, r0 + dr, pl.ds(m * L, L)])
                    lo.append(a)
                    hi.append(b)
                tlo = (lo[0] + lo[1]) + (lo[2] + lo[3])
                thi = (hi[0] + hi[1]) + (hi[2] + hi[3])
                out_ss.append(ss[2 * m] + tlo)
                out_ss.append(ss[2 * m + 1] + thi)
            return tuple(out_ss)

        init = tuple(jnp.zeros((L,), jnp.float32) for _ in range(2 * NBW))
        ss = lax.fori_loop(0, 2 * NA // 4, rbody, init)
        for m in range(NBW):
            hA = jnp.clip(ss[2 * m] + bias_v[pl.ds(m * L, L)], 0.0, 1.0)
            hB = jnp.clip(ss[2 * m + 1] + bias_v[pl.ds(W2 + m * L, L)], 0.0, 1.0)
            p = p + hA * w_v[pl.ds(half * D + m * L, L)]
            p = p + hB * w_v[pl.ds(half * D + W2 + m * L, L)]
        return p

    def hsum(v):
        for k in (8, 4, 2, 1):
            v = v + v.at[lane ^ k].get(mode="promise_in_bounds")
        return v

    def chunk_body(ci, carry):
        base = wid * BPW + ci * E

        pltpu.sync_copy(stm.at[pl.ds(base, E), :], i0)
        pltpu.sync_copy(nstm.at[pl.ds(base, E), :], i1)
        pltpu.sync_copy(fstm.at[pl.ds(base, E), :], i2)
        pltpu.sync_copy(fnstm.at[pl.ds(base, E), :], i3)

        fire2(0, 0, semA, i0, i2)

        def kbody(e, c):
            fire2(e, 1, semB, i1, i3)
            drain2(0, semA)
            p = reduce_half(0, 0, jnp.zeros((L,), jnp.float32))

            @pl.when(e < E - 1)
            def _():
                fire2(e + 1, 0, semA, i0, i2)

            drain2(1, semB)
            p = reduce_half(1, 1, p)
            pbuf[e, :] = p
            return c

        lax.fori_loop(0, E, kbody, 0)

        def gbody(gi, c):
            res = jnp.zeros((L,), jnp.float32)
            for l in range(L):
                pv = pbuf[gi * L + l, :]
                res = jnp.where(lane == l, hsum(pv), res)
            y = 1.0 / (1.0 + jnp.exp(-(res + outb_v[...])))
            outc[pl.ds(gi * L, L)] = y
            return c

        lax.fori_loop(0, E // L, gbody, 0)

        pltpu.sync_copy(outc, out.at[pl.ds(base, E)])
        return carry

    lax.fori_loop(0, NCHUNK, chunk_body, 0)


def _deinterleave(v):
    return jnp.concatenate(
        [v.reshape(NBW, L, 2)[:, :, 0].reshape(-1),
         v.reshape(NBW, L, 2)[:, :, 1].reshape(-1)])


def _pack(tbl):
    v, d = tbl.shape
    b = tbl.astype(jnp.bfloat16).reshape(v, d // 2, 2)
    return lax.bitcast_convert_type(b, jnp.int32)


def kernel(stm_idx, nstm_idx, f_stm_idx, f_nstm_idx,
           ft_kernel, ft_bias, fft_kernel, fft_bias, out_kernel, out_bias):
    mesh = plsc.VectorSubcoreMesh(core_axis_name="c", subcore_axis_name="s",
                                  num_cores=NC, num_subcores=NS)
    run = pl.kernel(
        _body,
        out_type=jax.ShapeDtypeStruct((B,), jnp.float32),
        mesh=mesh,
        compiler_params=pltpu.CompilerParams(needs_layout_passes=False),
        scratch_types=[
            pltpu.VMEM((E, NA), jnp.int32),
            pltpu.VMEM((E, NA), jnp.int32),
            pltpu.VMEM((E, NA), jnp.int32),
            pltpu.VMEM((E, NA), jnp.int32),
            pltpu.VMEM((2, 2 * NA, W2), jnp.int32),
            pltpu.VMEM((E, L), jnp.float32),
            pltpu.VMEM((D,), jnp.float32),
            pltpu.VMEM((2 * D,), jnp.float32),
            pltpu.VMEM((L,), jnp.float32),
            pltpu.VMEM((E,), jnp.float32),
            pltpu.SemaphoreType.DMA,
            pltpu.SemaphoreType.DMA,
        ],
    )
    bias01 = _deinterleave(ft_bias + fft_bias)
    w0 = _deinterleave(out_kernel[:D, 0])
    w1 = _deinterleave(out_kernel[D:, 0])
    wvec = jnp.concatenate([w0, w1])
    outb = jnp.broadcast_to(out_bias, (L,))
    y = run(stm_idx, nstm_idx, f_stm_idx, f_nstm_idx,
            _pack(ft_kernel), _pack(fft_kernel), bias01, wvec, outb)
    return y.reshape(B, 1)


# speedup vs baseline: 1.0155x; 1.0155x over previous
import jax
import jax.numpy as jnp
from jax import lax
from jax.experimental import pallas as pl
from jax.experimental.pallas import tpu as pltpu
from jax.experimental.pallas import tpu_sc as plsc

NC = 2
NS = 16
L = 16
NW = NC * NS

B = 16384
D = 256
W2 = D // 2
NA = 32
E = 128
BPW = B // NW
NCHUNK = BPW // E
NBW = W2 // L

def _body(stm, nstm, fstm, fnstm, ft, fft, bias, w, outb, out,
          i0, i1, i2, i3, rows, pbuf, bias_v, w_v, outb_v, outc,
          semA, semB):
    wid = lax.axis_index("s") * NC + lax.axis_index("c")

    pltpu.sync_copy(bias, bias_v)
    pltpu.sync_copy(w, w_v)
    pltpu.sync_copy(outb, outb_v)

    lane = lax.iota(jnp.int32, L)

    def fire2(e, par, sem, big, small):
        pltpu.async_copy(ft.at[big.at[e]], rows.at[par, pl.ds(0, NA)], sem)
        pltpu.async_copy(fft.at[small.at[e]], rows.at[par, pl.ds(NA, NA)], sem)

    def drain2(par, sem):
        for _ in range(2):
            pltpu.make_async_copy(
                ft.at[i0.at[0]], rows.at[par, pl.ds(0, NA)], sem).wait()

    def unpack(wv):
        lo = plsc.bitcast(lax.shift_left(wv, 16), jnp.float32)
        hi = plsc.bitcast(jnp.bitwise_and(wv, -65536), jnp.float32)
        return lo, hi

    def reduce_half(par, half, p):
        def rbody(r, ss):
            out_ss = []
            for m in range(NBW):
                lo, hi = unpack(rows[par, r, pl.ds(m * L, L)])
                out_ss.append(ss[2 * m] + lo)
                out_ss.append(ss[2 * m + 1] + hi)
            return tuple(out_ss)

        init = []
        for m in range(NBW):
            lo, hi = unpack(rows[par, 0, pl.ds(m * L, L)])
            init.extend([lo, hi])
        ss = lax.fori_loop(1, 2 * NA, rbody, tuple(init))
        for m in range(NBW):
            hA = jnp.clip(ss[2 * m] + bias_v[pl.ds(m * L, L)], 0.0, 1.0)
            hB = jnp.clip(ss[2 * m + 1] + bias_v[pl.ds(W2 + m * L, L)], 0.0, 1.0)
            p = p + hA * w_v[pl.ds(half * D + m * L, L)]
            p = p + hB * w_v[pl.ds(half * D + W2 + m * L, L)]
        return p

    def hsum(v):
        for k in (8, 4, 2, 1):
            v = v + v.at[lane ^ k].get(mode="promise_in_bounds")
        return v

    def chunk_body(ci, carry):
        base = wid * BPW + ci * E

        pltpu.sync_copy(stm.at[pl.ds(base, E), :], i0)
        pltpu.sync_copy(nstm.at[pl.ds(base, E), :], i1)
        pltpu.sync_copy(fstm.at[pl.ds(base, E), :], i2)
        pltpu.sync_copy(fnstm.at[pl.ds(base, E), :], i3)

        fire2(0, 0, semA, i0, i2)

        def kbody(e, c):
            fire2(e, 1, semB, i1, i3)
            drain2(0, semA)
            p = reduce_half(0, 0, jnp.zeros((L,), jnp.float32))

            @pl.when(e < E - 1)
            def _():
                fire2(e + 1, 0, semA, i0, i2)

            drain2(1, semB)
            p = reduce_half(1, 1, p)
            pbuf[e, :] = p
            return c

        lax.fori_loop(0, E, kbody, 0)

        def gbody(gi, c):
            res = jnp.zeros((L,), jnp.float32)
            for l in range(L):
                pv = pbuf[gi * L + l, :]
                res = jnp.where(lane == l, hsum(pv), res)
            y = 1.0 / (1.0 + jnp.exp(-(res + outb_v[...])))
            outc[pl.ds(gi * L, L)] = y
            return c

        lax.fori_loop(0, E // L, gbody, 0)

        pltpu.sync_copy(outc, out.at[pl.ds(base, E)])
        return carry

    lax.fori_loop(0, NCHUNK, chunk_body, 0)


def _deinterleave(v):
    return jnp.concatenate(
        [v.reshape(NBW, L, 2)[:, :, 0].reshape(-1),
         v.reshape(NBW, L, 2)[:, :, 1].reshape(-1)])


def _pack(tbl):
    v, d = tbl.shape
    b = tbl.astype(jnp.bfloat16).reshape(v, d // 2, 2)
    return lax.bitcast_convert_type(b, jnp.int32)


def kernel(stm_idx, nstm_idx, f_stm_idx, f_nstm_idx,
           ft_kernel, ft_bias, fft_kernel, fft_bias, out_kernel, out_bias):
    mesh = plsc.VectorSubcoreMesh(core_axis_name="c", subcore_axis_name="s",
                                  num_cores=NC, num_subcores=NS)
    run = pl.kernel(
        _body,
        out_type=jax.ShapeDtypeStruct((B,), jnp.float32),
        mesh=mesh,
        compiler_params=pltpu.CompilerParams(needs_layout_passes=False),
        scratch_types=[
            pltpu.VMEM((E, NA), jnp.int32),
            pltpu.VMEM((E, NA), jnp.int32),
            pltpu.VMEM((E, NA), jnp.int32),
            pltpu.VMEM((E, NA), jnp.int32),
            pltpu.VMEM((2, 2 * NA, W2), jnp.int32),
            pltpu.VMEM((E, L), jnp.float32),
            pltpu.VMEM((D,), jnp.float32),
            pltpu.VMEM((2 * D,), jnp.float32),
            pltpu.VMEM((L,), jnp.float32),
            pltpu.VMEM((E,), jnp.float32),
            pltpu.SemaphoreType.DMA,
            pltpu.SemaphoreType.DMA,
        ],
    )
    bias01 = _deinterleave(ft_bias + fft_bias)
    w0 = _deinterleave(out_kernel[:D, 0])
    w1 = _deinterleave(out_kernel[D:, 0])
    wvec = jnp.concatenate([w0, w1])
    outb = jnp.broadcast_to(out_bias, (L,))
    y = run(stm_idx, nstm_idx, f_stm_idx, f_nstm_idx,
            _pack(ft_kernel), _pack(fft_kernel), bias01, wvec, outb)
    return y.reshape(B, 1)


# speedup vs baseline: 1.1577x; 1.1400x over previous
import jax
import jax.numpy as jnp
from jax import lax
from jax.experimental import pallas as pl
from jax.experimental.pallas import tpu as pltpu
from jax.experimental.pallas import tpu_sc as plsc

NC = 2
NS = 16
L = 16
NW = NC * NS

B = 16384
D = 256
W2 = D // 2
NA = 32
E = 128
BPW = B // NW
NCHUNK = BPW // E
NBW = W2 // L
EP = E // 2


def _body(stm, nstm, fstm, fnstm, ft, fft, bias, w, outb, out,
          i0, i1, i2, i3, rows, pbuf, bias_v, w_v, outb_v, outc,
          semA, semB):
    wid = lax.axis_index("s") * NC + lax.axis_index("c")

    pltpu.sync_copy(bias, bias_v)
    pltpu.sync_copy(w, w_v)
    pltpu.sync_copy(outb, outb_v)

    lane = lax.iota(jnp.int32, L)

    def fire2(ep, par, sem, big, small):
        pltpu.async_copy(ft.at[big.at[ep]], rows.at[par, pl.ds(0, 2 * NA)],
                         sem)
        pltpu.async_copy(fft.at[small.at[ep]],
                         rows.at[par, pl.ds(2 * NA, 2 * NA)], sem)

    def drain2(par, sem):
        for _ in range(2):
            pltpu.make_async_copy(
                ft.at[i0.at[0]], rows.at[par, pl.ds(0, 2 * NA)], sem).wait()

    def unpack(wv):
        lo = plsc.bitcast(lax.shift_left(wv, 16), jnp.float32)
        hi = plsc.bitcast(jnp.bitwise_and(wv, -65536), jnp.float32)
        return lo, hi

    def add_rows(par, rbase, ss):
        def rbody(r, ss):
            out_ss = []
            for m in range(NBW):
                lo, hi = unpack(rows[par, r, pl.ds(m * L, L)])
                out_ss.append(ss[2 * m] + lo)
                out_ss.append(ss[2 * m + 1] + hi)
            return tuple(out_ss)

        return lax.fori_loop(rbase, rbase + NA, rbody, ss)

    def reduce_elem(par, sub, half, p):
        ss = tuple(jnp.zeros((L,), jnp.float32) for _ in range(2 * NBW))
        ss = add_rows(par, sub * NA, ss)
        ss = add_rows(par, 2 * NA + sub * NA, ss)
        for m in range(NBW):
            hA = jnp.clip(ss[2 * m] + bias_v[pl.ds(m * L, L)], 0.0, 1.0)
            hB = jnp.clip(ss[2 * m + 1] + bias_v[pl.ds(W2 + m * L, L)],
                          0.0, 1.0)
            p = p + hA * w_v[pl.ds(half * D + m * L, L)]
            p = p + hB * w_v[pl.ds(half * D + W2 + m * L, L)]
        return p

    def hsum(v):
        for k in (8, 4, 2, 1):
            v = v + v.at[lane ^ k].get(mode="promise_in_bounds")
        return v

    def chunk_body(ci, carry):
        base = wid * BPW + ci * E
        pbase = pl.multiple_of(base // 2, 64)

        pltpu.sync_copy(stm.at[pl.ds(pbase, EP), :], i0)
        pltpu.sync_copy(nstm.at[pl.ds(pbase, EP), :], i1)
        pltpu.sync_copy(fstm.at[pl.ds(pbase, EP), :], i2)
        pltpu.sync_copy(fnstm.at[pl.ds(pbase, EP), :], i3)

        fire2(0, 0, semA, i0, i2)

        def kbody(ep, c):
            fire2(ep, 1, semB, i1, i3)
            drain2(0, semA)
            zero = jnp.zeros((L,), jnp.float32)
            pA = reduce_elem(0, 0, 0, zero)
            pB = reduce_elem(0, 1, 0, zero)

            @pl.when(ep < EP - 1)
            def _():
                fire2(ep + 1, 0, semA, i0, i2)

            drain2(1, semB)
            pA = reduce_elem(1, 0, 1, pA)
            pB = reduce_elem(1, 1, 1, pB)
            pbuf[2 * ep, :] = pA
            pbuf[2 * ep + 1, :] = pB
            return c

        lax.fori_loop(0, EP, kbody, 0)

        def gbody(gi, c):
            res = jnp.zeros((L,), jnp.float32)
            for l in range(L):
                pv = pbuf[gi * L + l, :]
                res = jnp.where(lane == l, hsum(pv), res)
            y = 1.0 / (1.0 + jnp.exp(-(res + outb_v[...])))
            outc[pl.ds(gi * L, L)] = y
            return c

        lax.fori_loop(0, E // L, gbody, 0)

        pltpu.sync_copy(outc, out.at[pl.ds(base, E)])
        return carry

    lax.fori_loop(0, NCHUNK, chunk_body, 0)


def _deinterleave(v):
    return jnp.concatenate(
        [v.reshape(NBW, L, 2)[:, :, 0].reshape(-1),
         v.reshape(NBW, L, 2)[:, :, 1].reshape(-1)])


def _pack(tbl):
    v, d = tbl.shape
    b = tbl.astype(jnp.bfloat16).reshape(v, d // 2, 2)
    return lax.bitcast_convert_type(b, jnp.int32)


def kernel(stm_idx, nstm_idx, f_stm_idx, f_nstm_idx,
           ft_kernel, ft_bias, fft_kernel, fft_bias, out_kernel, out_bias):
    mesh = plsc.VectorSubcoreMesh(core_axis_name="c", subcore_axis_name="s",
                                  num_cores=NC, num_subcores=NS)
    run = pl.kernel(
        _body,
        out_type=jax.ShapeDtypeStruct((B,), jnp.float32),
        mesh=mesh,
        compiler_params=pltpu.CompilerParams(needs_layout_passes=False),
        scratch_types=[
            pltpu.VMEM((EP, 2 * NA), jnp.int32),
            pltpu.VMEM((EP, 2 * NA), jnp.int32),
            pltpu.VMEM((EP, 2 * NA), jnp.int32),
            pltpu.VMEM((EP, 2 * NA), jnp.int32),
            pltpu.VMEM((2, 4 * NA, W2), jnp.int32),
            pltpu.VMEM((E, L), jnp.float32),
            pltpu.VMEM((D,), jnp.float32),
            pltpu.VMEM((2 * D,), jnp.float32),
            pltpu.VMEM((L,), jnp.float32),
            pltpu.VMEM((E,), jnp.float32),
            pltpu.SemaphoreType.DMA,
            pltpu.SemaphoreType.DMA,
        ],
    )
    bias01 = _deinterleave(ft_bias + fft_bias)
    w0 = _deinterleave(out_kernel[:D, 0])
    w1 = _deinterleave(out_kernel[D:, 0])
    wvec = jnp.concatenate([w0, w1])
    outb = jnp.broadcast_to(out_bias, (L,))
    y = run(stm_idx.reshape(B // 2, 2 * NA), nstm_idx.reshape(B // 2, 2 * NA),
            f_stm_idx.reshape(B // 2, 2 * NA),
            f_nstm_idx.reshape(B // 2, 2 * NA),
            _pack(ft_kernel), _pack(fft_kernel), bias01, wvec, outb)
    return y.reshape(B, 1)


# speedup vs baseline: 1.1721x; 1.0125x over previous
import jax
import jax.numpy as jnp
from jax import lax
from jax.experimental import pallas as pl
from jax.experimental.pallas import tpu as pltpu
from jax.experimental.pallas import tpu_sc as plsc

NC = 2
NS = 16
L = 16
NW = NC * NS

B = 16384
D = 256
W2 = D // 2
NA = 32
NR = 2 * NA
E = 128
BPW = B // NW
NCHUNK = BPW // E
NBW = W2 // L
EP = E // 2


def _body(idx0, idx1, tbl, bias, w, outb, out,
          i0, i1, rows, pbuf, bias_v, w_v, outb_v, outc,
          semA, semB):
    wid = lax.axis_index("s") * NC + lax.axis_index("c")

    pltpu.sync_copy(bias, bias_v)
    pltpu.sync_copy(w, w_v)
    pltpu.sync_copy(outb, outb_v)

    lane = lax.iota(jnp.int32, L)

    def fire(ep, par, sem, idx):
        pltpu.async_copy(tbl.at[idx.at[ep]], rows.at[par], sem)

    def drain(par, sem):
        pltpu.make_async_copy(tbl.at[i0.at[0]], rows.at[par], sem).wait()

    def unpack(wv):
        lo = plsc.bitcast(lax.shift_left(wv, 16), jnp.float32)
        hi = plsc.bitcast(jnp.bitwise_and(wv, -65536), jnp.float32)
        return lo, hi

    def reduce_elem(par, sub, half, p):
        def rbody(r, ss):
            out_ss = []
            for m in range(NBW):
                lo, hi = unpack(rows[par, r, pl.ds(m * L, L)])
                out_ss.append(ss[2 * m] + lo)
                out_ss.append(ss[2 * m + 1] + hi)
            return tuple(out_ss)

        ss = tuple(jnp.zeros((L,), jnp.float32) for _ in range(2 * NBW))
        ss = lax.fori_loop(sub * NR, sub * NR + NR, rbody, ss)
        for m in range(NBW):
            hA = jnp.clip(ss[2 * m] + bias_v[pl.ds(m * L, L)], 0.0, 1.0)
            hB = jnp.clip(ss[2 * m + 1] + bias_v[pl.ds(W2 + m * L, L)],
                          0.0, 1.0)
            p = p + hA * w_v[pl.ds(half * D + m * L, L)]
            p = p + hB * w_v[pl.ds(half * D + W2 + m * L, L)]
        return p

    def hsum(v):
        for k in (8, 4, 2, 1):
            v = v + v.at[lane ^ k].get(mode="promise_in_bounds")
        return v

    def chunk_body(ci, carry):
        base = wid * BPW + ci * E
        pbase = pl.multiple_of(base // 2, 64)

        pltpu.sync_copy(idx0.at[pl.ds(pbase, EP), :], i0)
        pltpu.sync_copy(idx1.at[pl.ds(pbase, EP), :], i1)

        fire(0, 0, semA, i0)

        def kbody(ep, c):
            fire(ep, 1, semB, i1)
            drain(0, semA)
            zero = jnp.zeros((L,), jnp.float32)
            pA = reduce_elem(0, 0, 0, zero)
            pB = reduce_elem(0, 1, 0, zero)

            @pl.when(ep < EP - 1)
            def _():
                fire(ep + 1, 0, semA, i0)

            drain(1, semB)
            pA = reduce_elem(1, 0, 1, pA)
            pB = reduce_elem(1, 1, 1, pB)
            pbuf[2 * ep, :] = pA
            pbuf[2 * ep + 1, :] = pB
            return c

        lax.fori_loop(0, EP, kbody, 0)

        def gbody(gi, c):
            res = jnp.zeros((L,), jnp.float32)
            for l in range(L):
                pv = pbuf[gi * L + l, :]
                res = jnp.where(lane == l, hsum(pv), res)
            y = 1.0 / (1.0 + jnp.exp(-(res + outb_v[...])))
            outc[pl.ds(gi * L, L)] = y
            return c

        lax.fori_loop(0, E // L, gbody, 0)

        pltpu.sync_copy(outc, out.at[pl.ds(base, E)])
        return carry

    lax.fori_loop(0, NCHUNK, chunk_body, 0)


def _deinterleave(v):
    return jnp.concatenate(
        [v.reshape(NBW, L, 2)[:, :, 0].reshape(-1),
         v.reshape(NBW, L, 2)[:, :, 1].reshape(-1)])


def _pack(tbl):
    v, d = tbl.shape
    b = tbl.astype(jnp.bfloat16).reshape(v, d // 2, 2)
    return lax.bitcast_convert_type(b, jnp.int32)


def kernel(stm_idx, nstm_idx, f_stm_idx, f_nstm_idx,
           ft_kernel, ft_bias, fft_kernel, fft_bias, out_kernel, out_bias):
    mesh = plsc.VectorSubcoreMesh(core_axis_name="c", subcore_axis_name="s",
                                  num_cores=NC, num_subcores=NS)
    run = pl.kernel(
        _body,
        out_type=jax.ShapeDtypeStruct((B,), jnp.float32),
        mesh=mesh,
        compiler_params=pltpu.CompilerParams(needs_layout_passes=False),
        scratch_types=[
            pltpu.VMEM((EP, 2 * NR), jnp.int32),
            pltpu.VMEM((EP, 2 * NR), jnp.int32),
            pltpu.VMEM((2, 2 * NR, W2), jnp.int32),
            pltpu.VMEM((E, L), jnp.float32),
            pltpu.VMEM((D,), jnp.float32),
            pltpu.VMEM((2 * D,), jnp.float32),
            pltpu.VMEM((L,), jnp.float32),
            pltpu.VMEM((E,), jnp.float32),
            pltpu.SemaphoreType.DMA,
            pltpu.SemaphoreType.DMA,
        ],
    )
    nbig = ft_kernel.shape[0]
    tbl = jnp.concatenate([_pack(ft_kernel), _pack(fft_kernel)])
    idx0 = jnp.concatenate([stm_idx, f_stm_idx + nbig],
                           axis=1).reshape(B // 2, 2 * NR)
    idx1 = jnp.concatenate([nstm_idx, f_nstm_idx + nbig],
                           axis=1).reshape(B // 2, 2 * NR)
    bias01 = _deinterleave(ft_bias + fft_bias)
    w0 = _deinterleave(out_kernel[:D, 0])
    w1 = _deinterleave(out_kernel[D:, 0])
    wvec = jnp.concatenate([w0, w1])
    outb = jnp.broadcast_to(out_bias, (L,))
    y = run(idx0, idx1, tbl, bias01, wvec, outb)
    return y.reshape(B, 1)


# speedup vs baseline: 1.2107x; 1.0329x over previous
import jax
import jax.numpy as jnp
from jax import lax
from jax.experimental import pallas as pl
from jax.experimental.pallas import tpu as pltpu
from jax.experimental.pallas import tpu_sc as plsc

NC = 2
NS = 16
L = 16
NW = NC * NS

B = 16384
D = 256
W2 = D // 2
NA = 32
NR = 2 * NA
E = 128
BPW = B // NW
NCHUNK = BPW // E
NBW = W2 // L
EP = E // 2


def _body(idx0, idx1, tbl, bias, w, outb, out,
          i0, i1, rows, pbuf, bias_v, w_v, outb_v, outc,
          semA, semB):
    wid = lax.axis_index("s") * NC + lax.axis_index("c")

    pltpu.sync_copy(bias, bias_v)
    pltpu.sync_copy(w, w_v)
    pltpu.sync_copy(outb, outb_v)

    lane = lax.iota(jnp.int32, L)

    def fire(ep, par, sem, idx):
        pltpu.async_copy(tbl.at[idx.at[ep]], rows.at[par], sem)

    def drain(par, sem):
        pltpu.make_async_copy(tbl.at[i0.at[0]], rows.at[par], sem).wait()

    def unpack(wv):
        lo = plsc.bitcast(lax.shift_left(wv, 16), jnp.float32)
        hi = plsc.bitcast(jnp.bitwise_and(wv, -65536), jnp.float32)
        return lo, hi

    def reduce_elem(par, sub, half, p):
        def rbody(r, ss):
            return tuple(
                ss[m] + plsc.bitcast(rows[par, r, pl.ds(m * L, L)],
                                     jnp.bfloat16)
                for m in range(NBW))

        ss = tuple(jnp.zeros((2 * L,), jnp.bfloat16) for _ in range(NBW))
        ss = lax.fori_loop(sub * NR, sub * NR + NR, rbody, ss)
        for m in range(NBW):
            lo, hi = plsc.unpack(ss[m], format=plsc.PackFormat.INTERLEAVED,
                                 preferred_element_type=jnp.float32)
            hA = jnp.clip(lo + bias_v[pl.ds(m * L, L)], 0.0, 1.0)
            hB = jnp.clip(hi + bias_v[pl.ds(W2 + m * L, L)], 0.0, 1.0)
            p = p + hA * w_v[pl.ds(half * D + m * L, L)]
            p = p + hB * w_v[pl.ds(half * D + W2 + m * L, L)]
        return p

    def hsum(v):
        for k in (8, 4, 2, 1):
            v = v + v.at[lane ^ k].get(mode="promise_in_bounds")
        return v

    def chunk_body(ci, carry):
        base = wid * BPW + ci * E
        pbase = pl.multiple_of(base // 2, 64)

        pltpu.sync_copy(idx0.at[pl.ds(pbase, EP), :], i0)
        pltpu.sync_copy(idx1.at[pl.ds(pbase, EP), :], i1)

        fire(0, 0, semA, i0)

        def kbody(ep, c):
            fire(ep, 1, semB, i1)
            drain(0, semA)
            zero = jnp.zeros((L,), jnp.float32)
            pA = reduce_elem(0, 0, 0, zero)
            pB = reduce_elem(0, 1, 0, zero)

            @pl.when(ep < EP - 1)
            def _():
                fire(ep + 1, 0, semA, i0)

            drain(1, semB)
            pA = reduce_elem(1, 0, 1, pA)
            pB = reduce_elem(1, 1, 1, pB)
            pbuf[2 * ep, :] = pA
            pbuf[2 * ep + 1, :] = pB
            return c

        lax.fori_loop(0, EP, kbody, 0)

        def gbody(gi, c):
            res = jnp.zeros((L,), jnp.float32)
            for l in range(L):
                pv = pbuf[gi * L + l, :]
                res = jnp.where(lane == l, hsum(pv), res)
            y = 1.0 / (1.0 + jnp.exp(-(res + outb_v[...])))
            outc[pl.ds(gi * L, L)] = y
            return c

        lax.fori_loop(0, E // L, gbody, 0)

        pltpu.sync_copy(outc, out.at[pl.ds(base, E)])
        return carry

    lax.fori_loop(0, NCHUNK, chunk_body, 0)


def _deinterleave(v):
    return jnp.concatenate(
        [v.reshape(NBW, L, 2)[:, :, 0].reshape(-1),
         v.reshape(NBW, L, 2)[:, :, 1].reshape(-1)])


def _pack(tbl):
    v, d = tbl.shape
    b = tbl.astype(jnp.bfloat16).reshape(v, d // 2, 2)
    return lax.bitcast_convert_type(b, jnp.int32)


def kernel(stm_idx, nstm_idx, f_stm_idx, f_nstm_idx,
           ft_kernel, ft_bias, fft_kernel, fft_bias, out_kernel, out_bias):
    mesh = plsc.VectorSubcoreMesh(core_axis_name="c", subcore_axis_name="s",
                                  num_cores=NC, num_subcores=NS)
    run = pl.kernel(
        _body,
        out_type=jax.ShapeDtypeStruct((B,), jnp.float32),
        mesh=mesh,
        compiler_params=pltpu.CompilerParams(needs_layout_passes=False),
        scratch_types=[
            pltpu.VMEM((EP, 2 * NR), jnp.int32),
            pltpu.VMEM((EP, 2 * NR), jnp.int32),
            pltpu.VMEM((2, 2 * NR, W2), jnp.int32),
            pltpu.VMEM((E, L), jnp.float32),
            pltpu.VMEM((D,), jnp.float32),
            pltpu.VMEM((2 * D,), jnp.float32),
            pltpu.VMEM((L,), jnp.float32),
            pltpu.VMEM((E,), jnp.float32),
            pltpu.SemaphoreType.DMA,
            pltpu.SemaphoreType.DMA,
        ],
    )
    nbig = ft_kernel.shape[0]
    tbl = jnp.concatenate([_pack(ft_kernel), _pack(fft_kernel)])
    idx0 = jnp.concatenate([stm_idx, f_stm_idx + nbig],
                           axis=1).reshape(B // 2, 2 * NR)
    idx1 = jnp.concatenate([nstm_idx, f_nstm_idx + nbig],
                           axis=1).reshape(B // 2, 2 * NR)
    bias01 = _deinterleave(ft_bias + fft_bias)
    w0 = _deinterleave(out_kernel[:D, 0])
    w1 = _deinterleave(out_kernel[D:, 0])
    wvec = jnp.concatenate([w0, w1])
    outb = jnp.broadcast_to(out_bias, (L,))
    y = run(idx0, idx1, tbl, bias01, wvec, outb)
    return y.reshape(B, 1)
